# Initial kernel scaffold; baseline (speedup 1.0000x reference)
import jax
import jax.numpy as jnp
from jax.experimental import pallas as pl


def kernel(in_feat, edge_index, lin_w1, lin_b1, w_ih1, w_hh1, b_ih1, b_hh1, lin_w2, lin_b2, w_ih2, w_hh2, b_ih2, b_hh2):
    raise NotImplementedError("write your pallas kernel here")


# speedup vs baseline: 3.6273x; 3.6273x over previous
import functools

import jax
import jax.numpy as jnp
from jax import lax
from jax.experimental import pallas as pl
from jax.experimental.pallas import tpu as pltpu
from jax.experimental.pallas import tpu_sc as plsc

N_NODES = 10000
DIM = 128
N_STEPS = 8

NCORES = 2
NSUB = 16
NWORK = NCORES * NSUB
CHUNK = 128
NPAD = 10112

ROWS = 1000


def _make_segsum(e_pad):
    ept = e_pad // NWORK
    nchunk = ept // CHUNK
    zrows = NPAD // NSUB
    mesh = plsc.VectorSubcoreMesh(core_axis_name="c", subcore_axis_name="s")

    @functools.partial(
        pl.kernel,
        out_type=jax.ShapeDtypeStruct((NCORES, NPAD, DIM), jnp.float32),
        mesh=mesh,
        scratch_types=[
            pltpu.VMEM((CHUNK,), jnp.int32),
            pltpu.VMEM((CHUNK,), jnp.int32),
            pltpu.VMEM((CHUNK, DIM), jnp.float32),
            pltpu.VMEM_SHARED((NPAD, DIM), jnp.float32),
            pltpu.SemaphoreType.DMA,
        ],
    )
    def segsum(hw_hbm, src_hbm, dst_hbm, zeros_hbm, out_hbm,
               src_v, dst_v, rows_v, acc_sh, sem):
        cid = lax.axis_index("c")
        sid = lax.axis_index("s")
        wid = cid * NSUB + sid
        pltpu.sync_copy(zeros_hbm.at[pl.ds(sid * zrows, zrows)],
                        acc_sh.at[pl.ds(sid * zrows, zrows)])
        plsc.subcore_barrier()

        def body(c, carry):
            base = pl.multiple_of(wid * ept + c * CHUNK, CHUNK)
            pltpu.sync_copy(src_hbm.at[pl.ds(base, CHUNK)], src_v)
            pltpu.sync_copy(dst_hbm.at[pl.ds(base, CHUNK)], dst_v)
            pltpu.async_copy(hw_hbm.at[src_v], rows_v, sem).wait()
            pltpu.sync_copy(rows_v, acc_sh.at[dst_v], add=True)
            return carry

        lax.fori_loop(0, nchunk, body, 0)
        plsc.subcore_barrier()
        pltpu.sync_copy(acc_sh.at[pl.ds(sid * zrows, zrows)],
                        out_hbm.at[cid, pl.ds(sid * zrows, zrows)])

    return segsum


def _dotT(x, w):
    return lax.dot_general(x, w, (((1,), (1,)), ((), ())),
                           preferred_element_type=jnp.float32)


_row_spec = pl.BlockSpec((ROWS, DIM), lambda i: (i, 0))
_row3_spec = pl.BlockSpec((ROWS, 3 * DIM), lambda i: (i, 0))
_parts_spec = pl.BlockSpec((NCORES, ROWS, DIM), lambda i: (0, i, 0))


def _full(shape):
    return pl.BlockSpec(shape, lambda i: tuple(0 for _ in shape))


def _tc_in(h, lw, lb, whh, bhh):
    def body(h_ref, lw_ref, lb_ref, whh_ref, bhh_ref, hw_ref, gh_ref):
        h_blk = h_ref[...]
        hw_ref[...] = _dotT(h_blk, lw_ref[...]) + lb_ref[...]
        gh_ref[...] = _dotT(h_blk, whh_ref[...]) + bhh_ref[...]

    grid = (N_NODES // ROWS,)
    return pl.pallas_call(
        body,
        grid=grid,
        in_specs=[_row_spec, _full((DIM, DIM)), _full((1, DIM)),
                  _full((3 * DIM, DIM)), _full((1, 3 * DIM))],
        out_specs=[_row_spec, _row3_spec],
        out_shape=[jax.ShapeDtypeStruct((N_NODES, DIM), jnp.float32),
                   jax.ShapeDtypeStruct((N_NODES, 3 * DIM), jnp.float32)],
    )(h, lw, lb, whh, bhh)


def _gru_core(a, h, gh, wih_ref, bih_ref):
    gi = _dotT(a, wih_ref[...]) + bih_ref[...]
    i_r = gi[:, :DIM]
    i_z = gi[:, DIM:2 * DIM]
    i_n = gi[:, 2 * DIM:]
    h_r = gh[:, :DIM]
    h_z = gh[:, DIM:2 * DIM]
    h_n = gh[:, 2 * DIM:]
    r = jax.nn.sigmoid(i_r + h_r)
    z = jax.nn.sigmoid(i_z + h_z)
    ng = jnp.tanh(i_n + r * h_n)
    return (1.0 - z) * ng + z * h


def _tc_step(parts, h, gh, wih, bih, lw, lb, whh, bhh, relu):
    def body(parts_ref, h_ref, gh_ref, wih_ref, bih_ref,
             lw_ref, lb_ref, whh_ref, bhh_ref, h_out_ref, hw_ref, gh_out_ref):
        a = parts_ref[0] + parts_ref[1]
        hn = _gru_core(a, h_ref[...], gh_ref[...], wih_ref, bih_ref)
        if relu:
            hn = jnp.maximum(hn, 0.0)
        h_out_ref[...] = hn
        hw_ref[...] = _dotT(hn, lw_ref[...]) + lb_ref[...]
        gh_out_ref[...] = _dotT(hn, whh_ref[...]) + bhh_ref[...]

    grid = (N_NODES // ROWS,)
    return pl.pallas_call(
        body,
        grid=grid,
        in_specs=[_parts_spec, _row_spec, _row3_spec,
                  _full((3 * DIM, DIM)), _full((1, 3 * DIM)),
                  _full((DIM, DIM)), _full((1, DIM)),
                  _full((3 * DIM, DIM)), _full((1, 3 * DIM))],
        out_specs=[_row_spec, _row_spec, _row3_spec],
        out_shape=[jax.ShapeDtypeStruct((N_NODES, DIM), jnp.float32),
                   jax.ShapeDtypeStruct((N_NODES, DIM), jnp.float32),
                   jax.ShapeDtypeStruct((N_NODES, 3 * DIM), jnp.float32)],
    )(parts, h, gh, wih, bih, lw, lb, whh, bhh)


def _tc_step_mean(parts, h, gh, wih, bih):
    def body(parts_ref, h_ref, gh_ref, wih_ref, bih_ref, out_ref):
        a = parts_ref[0] + parts_ref[1]
        hn = _gru_core(a, h_ref[...], gh_ref[...], wih_ref, bih_ref)

        @pl.when(pl.program_id(0) == 0)
        def _():
            out_ref[...] = jnp.zeros_like(out_ref)

        out_ref[...] += jnp.sum(hn, axis=0, keepdims=True) * (1.0 / N_NODES)

    grid = (N_NODES // ROWS,)
    return pl.pallas_call(
        body,
        grid=grid,
        in_specs=[_parts_spec, _row_spec, _row3_spec,
                  _full((3 * DIM, DIM)), _full((1, 3 * DIM))],
        out_specs=pl.BlockSpec((1, DIM), lambda i: (0, 0)),
        out_shape=jax.ShapeDtypeStruct((1, DIM), jnp.float32),
    )(parts, h, gh, wih, bih)


def kernel(in_feat, edge_index, lin_w1, lin_b1, w_ih1, w_hh1, b_ih1, b_hh1,
           lin_w2, lin_b2, w_ih2, w_hh2, b_ih2, b_hh2):
    src = edge_index[0]
    dst = edge_index[1]
    e = src.shape[0]
    step = NWORK * CHUNK
    e_pad = ((e + step - 1) // step) * step
    pad = e_pad - e
    src_p = jnp.concatenate([src, jnp.zeros((pad,), jnp.int32)])
    dst_p = jnp.concatenate([dst, jnp.full((pad,), N_NODES, jnp.int32)])
    zeros = jnp.zeros((NPAD, DIM), jnp.float32)

    segsum = _make_segsum(e_pad)

    lb1 = lin_b1.reshape(1, DIM)
    lb2 = lin_b2.reshape(1, DIM)
    bih1 = b_ih1.reshape(1, 3 * DIM)
    bih2 = b_ih2.reshape(1, 3 * DIM)
    bhh1 = b_hh1.reshape(1, 3 * DIM)
    bhh2 = b_hh2.reshape(1, 3 * DIM)

    h = in_feat
    hw, gh = _tc_in(h, lin_w1, lb1, w_hh1, bhh1)
    for t in range(N_STEPS - 1):
        parts = segsum(hw, src_p, dst_p, zeros)
        h, hw, gh = _tc_step(parts, h, gh, w_ih1, bih1,
                             lin_w1, lb1, w_hh1, bhh1, relu=False)
    parts = segsum(hw, src_p, dst_p, zeros)
    h, hw, gh = _tc_step(parts, h, gh, w_ih1, bih1,
                         lin_w2, lb2, w_hh2, bhh2, relu=True)
    for t in range(N_STEPS - 1):
        parts = segsum(hw, src_p, dst_p, zeros)
        h, hw, gh = _tc_step(parts, h, gh, w_ih2, bih2,
                             lin_w2, lb2, w_hh2, bhh2, relu=False)
    parts = segsum(hw, src_p, dst_p, zeros)
    return _tc_step_mean(parts, h, gh, w_ih2, bih2)
